# Initial kernel scaffold; baseline (speedup 1.0000x reference)
import functools

import jax
import jax.numpy as jnp
from jax import lax
from jax.experimental import pallas as pl
from jax.experimental.pallas import tpu as pltpu
from jax.experimental.pallas import tpu_sc as plsc

N = 10000
E = 320000
D = 128
NC = 2
NS = 16
NW = NC * NS
EPT = E // NW
CHUNK = 80
NCHUNK = EPT // CHUNK
RPT = N // NS
ZROWS = 125
DW = 16

_MESH = plsc.VectorSubcoreMesh(core_axis_name="c", subcore_axis_name="s")


@functools.partial(
    pl.kernel,
    out_type=jax.ShapeDtypeStruct((NC * N, DW), jnp.float32),
    mesh=_MESH,
    scratch_types=[
        pltpu.VMEM((NCHUNK, CHUNK), jnp.int32),
        pltpu.VMEM((CHUNK, DW), jnp.float32),
        pltpu.VMEM((ZROWS, DW), jnp.float32),
        pltpu.VMEM_SHARED((N, DW), jnp.float32),
    ],
)
def _sc_degree(dst_hbm, out_hbm, dst_v, ones_v, zbuf_v, acc_sh):
    c = lax.axis_index("c")
    s = lax.axis_index("s")
    wid = c * NS + s

    @pl.loop(0, ZROWS)
    def _(r):
        zbuf_v[r, :] = jnp.zeros((DW,), jnp.float32)

    @pl.loop(0, CHUNK)
    def _(r):
        ones_v[r, :] = jnp.full((DW,), 1.0, jnp.float32)

    r0 = s * RPT

    @pl.loop(0, RPT // ZROWS)
    def _(i):
        pltpu.sync_copy(zbuf_v, acc_sh.at[pl.ds(r0 + i * ZROWS, ZROWS)])

    pltpu.sync_copy(dst_hbm.at[wid], dst_v)
    plsc.subcore_barrier()

    @pl.loop(0, NCHUNK)
    def _(j):
        pltpu.sync_copy(ones_v, acc_sh.at[dst_v.at[j]], add=True)

    plsc.subcore_barrier()
    pltpu.sync_copy(acc_sh.at[pl.ds(r0, RPT)], out_hbm.at[pl.ds(c * N + r0, RPT)])


@functools.partial(
    pl.kernel,
    out_type=jax.ShapeDtypeStruct((NC * N, D), jnp.float32),
    mesh=_MESH,
    scratch_types=[
        pltpu.VMEM((NCHUNK, CHUNK), jnp.int32),
        pltpu.VMEM((NCHUNK, CHUNK), jnp.int32),
        pltpu.VMEM((CHUNK, D), jnp.float32),
        pltpu.VMEM((ZROWS, D), jnp.float32),
        pltpu.VMEM_SHARED((N, D), jnp.float32),
    ],
)
def _sc_aggregate(y_hbm, src_hbm, dst_hbm, out_hbm,
                  src_v, dst_v, rows_v, zbuf_v, acc_sh):
    c = lax.axis_index("c")
    s = lax.axis_index("s")
    wid = c * NS + s

    @pl.loop(0, ZROWS)
    def _(r):
        @pl.loop(0, D, step=16)
        def _(c0):
            zbuf_v[r, pl.ds(c0, 16)] = jnp.zeros((16,), jnp.float32)

    r0 = s * RPT

    @pl.loop(0, RPT // ZROWS)
    def _(i):
        pltpu.sync_copy(zbuf_v, acc_sh.at[pl.ds(r0 + i * ZROWS, ZROWS)])

    pltpu.sync_copy(src_hbm.at[wid], src_v)
    pltpu.sync_copy(dst_hbm.at[wid], dst_v)
    plsc.subcore_barrier()

    @pl.loop(0, NCHUNK)
    def _(j):
        pltpu.sync_copy(y_hbm.at[src_v.at[j]], rows_v)
        pltpu.sync_copy(rows_v, acc_sh.at[dst_v.at[j]], add=True)

    plsc.subcore_barrier()
    pltpu.sync_copy(acc_sh.at[pl.ds(r0, RPT)], out_hbm.at[pl.ds(c * N + r0, RPT)])


BM = 1000

_DOT = dict(precision=lax.Precision.HIGHEST, preferred_element_type=jnp.float32)


def _mm_body(x_ref, w_ref, o_ref):
    o_ref[...] = lax.dot_general(x_ref[...], w_ref[...],
                                 (((1,), (1,)), ((), ())), **_DOT)


def _matmul(x, w):
    return pl.pallas_call(
        _mm_body,
        grid=(N // BM,),
        in_specs=[pl.BlockSpec((BM, D), lambda i: (i, 0)),
                  pl.BlockSpec((D, D), lambda i: (0, 0))],
        out_specs=pl.BlockSpec((BM, D), lambda i: (i, 0)),
        out_shape=jax.ShapeDtypeStruct((N, D), jnp.float32),
    )(x, w)


def _dis_y_body(degp_ref, t_ref, dis_ref, y_ref):
    deg = degp_ref[0, :, 0] + degp_ref[1, :, 0] + 1.0
    dis = lax.rsqrt(deg)
    dis_ref[...] = dis
    y_ref[...] = t_ref[...] * dis[:, None]


def _dis_y(degp, t):
    return pl.pallas_call(
        _dis_y_body,
        grid=(N // BM,),
        in_specs=[pl.BlockSpec((NC, BM, DW), lambda i: (0, i, 0)),
                  pl.BlockSpec((BM, D), lambda i: (i, 0))],
        out_specs=[pl.BlockSpec((BM,), lambda i: (i,)),
                   pl.BlockSpec((BM, D), lambda i: (i, 0))],
        out_shape=[jax.ShapeDtypeStruct((N,), jnp.float32),
                   jax.ShapeDtypeStruct((N, D), jnp.float32)],
    )(degp, t)


def _mid_body(sp_ref, y_ref, dis_ref, b_ref, w_ref, o_ref):
    dis = dis_ref[...]
    agg = sp_ref[0] + sp_ref[1] + y_ref[...]
    h = jnp.maximum(agg * dis[:, None] + b_ref[...][None, :], 0.0)
    t = lax.dot_general(h, w_ref[...], (((1,), (1,)), ((), ())), **_DOT)
    o_ref[...] = t * dis[:, None]


def _mid_layer(sp, y, dis, b, w):
    return pl.pallas_call(
        _mid_body,
        grid=(N // BM,),
        in_specs=[pl.BlockSpec((NC, BM, D), lambda i: (0, i, 0)),
                  pl.BlockSpec((BM, D), lambda i: (i, 0)),
                  pl.BlockSpec((BM,), lambda i: (i,)),
                  pl.BlockSpec((D,), lambda i: (0,)),
                  pl.BlockSpec((D, D), lambda i: (0, 0))],
        out_specs=pl.BlockSpec((BM, D), lambda i: (i, 0)),
        out_shape=jax.ShapeDtypeStruct((N, D), jnp.float32),
    )(sp, y, dis, b, w)


def _final_body(sp_ref, y_ref, dis_ref, b_ref, o_ref):
    agg = sp_ref[0] + sp_ref[1] + y_ref[...]
    o_ref[...] = jnp.maximum(agg * dis_ref[...][:, None] + b_ref[...][None, :],
                             0.0)


def _final_layer(sp, y, dis, b):
    return pl.pallas_call(
        _final_body,
        grid=(N // BM,),
        in_specs=[pl.BlockSpec((NC, BM, D), lambda i: (0, i, 0)),
                  pl.BlockSpec((BM, D), lambda i: (i, 0)),
                  pl.BlockSpec((BM,), lambda i: (i,)),
                  pl.BlockSpec((D,), lambda i: (0,))],
        out_specs=pl.BlockSpec((BM, D), lambda i: (i, 0)),
        out_shape=jax.ShapeDtypeStruct((N, D), jnp.float32),
    )(sp, y, dis, b)


@jax.jit
def kernel(x, edge_index, W1, b1, W2, b2):
    ei = edge_index.astype(jnp.int32)
    src = ei[0].reshape(NW, NCHUNK, CHUNK)
    dst = ei[1].reshape(NW, NCHUNK, CHUNK)

    degp = _sc_degree(dst).reshape(NC, N, DW)
    t1 = _matmul(x, W1)
    dis, y1 = _dis_y(degp, t1)

    s1 = _sc_aggregate(y1, src, dst).reshape(NC, N, D)
    y2 = _mid_layer(s1, y1, dis, b1, W2)

    s2 = _sc_aggregate(y2, src, dst).reshape(NC, N, D)
    return _final_layer(s2, y2, dis, b2)


# speedup vs baseline: 20.0069x; 20.0069x over previous
import dataclasses
import functools

import jax
import jax.numpy as jnp
from jax import lax
from jax.experimental import pallas as pl
from jax.experimental.pallas import tpu as pltpu
from jax.experimental.pallas import tpu_sc as plsc

N = 10000
E = 320000
D = 128
NC = 2
NS = 16
NW = NC * NS
EPT = E // NW
CHUNK = 80
NCHUNK = EPT // CHUNK
NP = 10112
RPT = NP // NS
ZROWS = 8
L = 16

_MESH = plsc.VectorSubcoreMesh(core_axis_name="c", subcore_axis_name="s")

_SC_PARAMS = pltpu.CompilerParams()
if "needs_layout_passes" in pltpu.CompilerParams.__dataclass_fields__:
    _SC_PARAMS = dataclasses.replace(_SC_PARAMS, needs_layout_passes=False)


@functools.partial(
    pl.kernel,
    out_type=jax.ShapeDtypeStruct((NW, NP), jnp.float32),
    mesh=_MESH,
    scratch_types=[
        pltpu.VMEM((NCHUNK, CHUNK), jnp.int32),
        pltpu.VMEM((NP,), jnp.float32),
    ],
    compiler_params=_SC_PARAMS,
)
def _sc_degree(dst_hbm, out_hbm, dst_v, cnt_v):
    c = lax.axis_index("c")
    s = lax.axis_index("s")
    wid = c * NS + s

    @pl.loop(0, NP, step=L)
    def _(r):
        cnt_v[pl.ds(r, L)] = jnp.zeros((L,), jnp.float32)

    pltpu.sync_copy(dst_hbm.at[wid], dst_v)

    ones = jnp.full((L,), 1.0, jnp.float32)

    @pl.loop(0, NCHUNK)
    def _(j):
        @pl.loop(0, CHUNK, step=L)
        def _(g):
            idx = dst_v[j, pl.ds(g, L)]
            plsc.addupdate_scatter(cnt_v, [idx], ones)

    pltpu.sync_copy(cnt_v, out_hbm.at[wid])


@functools.partial(
    pl.kernel,
    out_type=jax.ShapeDtypeStruct((NC * NP, D), jnp.float32),
    mesh=_MESH,
    scratch_types=[
        pltpu.VMEM((NCHUNK, CHUNK), jnp.int32),
        pltpu.VMEM((NCHUNK, CHUNK), jnp.int32),
        pltpu.VMEM((CHUNK, D), jnp.float32),
        pltpu.VMEM((ZROWS, D), jnp.float32),
        pltpu.VMEM_SHARED((NP, D), jnp.float32),
    ],
)
def _sc_aggregate(y_hbm, src_hbm, dst_hbm, out_hbm,
                  src_v, dst_v, rows_v, zbuf_v, acc_sh):
    c = lax.axis_index("c")
    s = lax.axis_index("s")
    wid = c * NS + s

    @pl.loop(0, ZROWS)
    def _(r):
        @pl.loop(0, D, step=L)
        def _(c0):
            zbuf_v[r, pl.ds(c0, L)] = jnp.zeros((L,), jnp.float32)

    r0 = s * RPT

    @pl.loop(0, RPT, step=ZROWS)
    def _(i):
        pltpu.sync_copy(zbuf_v, acc_sh.at[pl.ds(r0 + i, ZROWS)])

    pltpu.sync_copy(src_hbm.at[wid], src_v)
    pltpu.sync_copy(dst_hbm.at[wid], dst_v)
    plsc.subcore_barrier()

    @pl.loop(0, NCHUNK)
    def _(j):
        pltpu.sync_copy(y_hbm.at[src_v.at[j]], rows_v)
        pltpu.sync_copy(rows_v, acc_sh.at[dst_v.at[j]], add=True)

    plsc.subcore_barrier()
    pltpu.sync_copy(acc_sh.at[pl.ds(r0, RPT)], out_hbm.at[pl.ds(c * NP + r0, RPT)])


BM = 1024

_DOT = dict(precision=lax.Precision.HIGHEST, preferred_element_type=jnp.float32)


def _mm_body(x_ref, w_ref, o_ref):
    o_ref[...] = lax.dot_general(x_ref[...], w_ref[...],
                                 (((1,), (1,)), ((), ())), **_DOT)


def _matmul(x, w):
    return pl.pallas_call(
        _mm_body,
        grid=(NP // BM + 1,),
        in_specs=[pl.BlockSpec((BM, D), lambda i: (i, 0)),
                  pl.BlockSpec((D, D), lambda i: (0, 0))],
        out_specs=pl.BlockSpec((BM, D), lambda i: (i, 0)),
        out_shape=jax.ShapeDtypeStruct((N, D), jnp.float32),
    )(x, w)


def _dis_y_body(degp_ref, t_ref, dis_ref, y_ref):
    deg = jnp.sum(degp_ref[...], axis=0) + 1.0
    dis = lax.rsqrt(deg)[:, None]
    dis_ref[...] = dis
    y_ref[...] = t_ref[...] * dis


def _dis_y(degp, t):
    return pl.pallas_call(
        _dis_y_body,
        grid=(NP // BM + 1,),
        in_specs=[pl.BlockSpec((NW, BM), lambda i: (0, i)),
                  pl.BlockSpec((BM, D), lambda i: (i, 0))],
        out_specs=[pl.BlockSpec((BM, 1), lambda i: (i, 0)),
                   pl.BlockSpec((BM, D), lambda i: (i, 0))],
        out_shape=[jax.ShapeDtypeStruct((NP, 1), jnp.float32),
                   jax.ShapeDtypeStruct((N, D), jnp.float32)],
    )(degp, t)


def _mid_body(sp_ref, y_ref, dis_ref, b_ref, w_ref, o_ref):
    dis = dis_ref[...]
    agg = sp_ref[0] + sp_ref[1] + y_ref[...]
    h = jnp.maximum(agg * dis + b_ref[...][None, :], 0.0)
    t = lax.dot_general(h, w_ref[...], (((1,), (1,)), ((), ())), **_DOT)
    o_ref[...] = t * dis


def _mid_layer(sp, y, dis, b, w):
    return pl.pallas_call(
        _mid_body,
        grid=(NP // BM + 1,),
        in_specs=[pl.BlockSpec((NC, BM, D), lambda i: (0, i, 0)),
                  pl.BlockSpec((BM, D), lambda i: (i, 0)),
                  pl.BlockSpec((BM, 1), lambda i: (i, 0)),
                  pl.BlockSpec((D,), lambda i: (0,)),
                  pl.BlockSpec((D, D), lambda i: (0, 0))],
        out_specs=pl.BlockSpec((BM, D), lambda i: (i, 0)),
        out_shape=jax.ShapeDtypeStruct((N, D), jnp.float32),
    )(sp, y, dis, b, w)


def _final_body(sp_ref, y_ref, dis_ref, b_ref, o_ref):
    agg = sp_ref[0] + sp_ref[1] + y_ref[...]
    o_ref[...] = jnp.maximum(agg * dis_ref[...] + b_ref[...][None, :], 0.0)


def _final_layer(sp, y, dis, b):
    return pl.pallas_call(
        _final_body,
        grid=(NP // BM + 1,),
        in_specs=[pl.BlockSpec((NC, BM, D), lambda i: (0, i, 0)),
                  pl.BlockSpec((BM, D), lambda i: (i, 0)),
                  pl.BlockSpec((BM, 1), lambda i: (i, 0)),
                  pl.BlockSpec((D,), lambda i: (0,))],
        out_specs=pl.BlockSpec((BM, D), lambda i: (i, 0)),
        out_shape=jax.ShapeDtypeStruct((N, D), jnp.float32),
    )(sp, y, dis, b)


@jax.jit
def kernel(x, edge_index, W1, b1, W2, b2):
    ei = edge_index.astype(jnp.int32)
    src = ei[0].reshape(NW, NCHUNK, CHUNK)
    dst = ei[1].reshape(NW, NCHUNK, CHUNK)

    degp = _sc_degree(dst)
    t1 = _matmul(x, W1)
    dis, y1 = _dis_y(degp, t1)

    s1 = _sc_aggregate(y1, src, dst).reshape(NC, NP, D)
    y2 = _mid_layer(s1, y1, dis, b1, W2)

    s2 = _sc_aggregate(y2, src, dst).reshape(NC, NP, D)
    return _final_layer(s2, y2, dis, b2)
